# baseline (speedup 1.0000x reference)
import functools

import numpy as np
import jax
import jax.numpy as jnp
from jax import lax
from jax.experimental import pallas as pl
from jax.experimental.pallas import tpu as pltpu
from jax.experimental.pallas import tpu_sc as plsc

VOCAB = 50000
EMB = 300
HID = 128
LSTMH = 128
NFEAT = 128
FEAT = 50
FFN = 512
NITER = 1
NW = 20000
NS = 1024
E = 131072
NDOC = 32
SPD = 32
L = 50
POSMAX = 51

NCORES = 2
NSUBCORES = 16
NTILES = NCORES * NSUBCORES
NSUB = 128


def _pos_table():
    pos = np.arange(POSMAX)[:, None].astype(np.float64)
    i = np.arange(EMB)[None, :]
    angle = pos / np.power(10000.0, 2.0 * (i // 2) / EMB)
    tab = np.zeros((POSMAX, EMB), dtype=np.float32)
    tab[:, 0::2] = np.sin(angle[:, 0::2])
    tab[:, 1::2] = np.cos(angle[:, 1::2])
    tab[0, :] = 0.0
    return jnp.asarray(tab)


def _lstm_dir(x, Wih, Whh, b, reverse):
    B, T, D = x.shape
    H = Whh.shape[0]
    xs = jnp.swapaxes(x, 0, 1)
    if reverse:
        xs = xs[::-1]

    def step(carry, xt):
        h, c = carry
        g = xt @ Wih + h @ Whh + b
        i, f, gg, o = jnp.split(g, 4, axis=-1)
        c = jax.nn.sigmoid(f) * c + jax.nn.sigmoid(i) * jnp.tanh(gg)
        h = jax.nn.sigmoid(o) * jnp.tanh(c)
        return (h, c), h

    _, hs = jax.lax.scan(step, (jnp.zeros((B, H), x.dtype), jnp.zeros((B, H), x.dtype)), xs)
    if reverse:
        hs = hs[::-1]
    return jnp.swapaxes(hs, 0, 1)


@functools.lru_cache(maxsize=None)
def _edge_gat_kernel(n_src, n_dst, nh, dh, P, NB):
    EPT = E // NTILES
    NBLK = EPT // NB
    NSUBS = NB // NSUB
    NG = NB // 16

    mesh = plsc.VectorSubcoreMesh(core_axis_name="c", subcore_axis_name="s")

    @functools.partial(
        pl.kernel,
        out_type=jax.ShapeDtypeStruct((nh, 2, n_dst, P), jnp.float32),
        mesh=mesh,
        compiler_params=pltpu.CompilerParams(
            use_tc_tiling_on_sc=False, needs_layout_passes=False),
        scratch_types=[
            pltpu.VMEM((NB, P), jnp.float32),
            pltpu.VMEM((NB,), jnp.int32),
            pltpu.VMEM((NSUBS, NSUB), jnp.int32),
            pltpu.VMEM((NB,), jnp.int32),
            pltpu.VMEM((NB,), jnp.int32),
            pltpu.VMEM((n_dst,), jnp.float32),
            pltpu.VMEM((16,), jnp.float32),
            pltpu.VMEM((16,), jnp.float32),
            pltpu.VMEM_SHARED((n_dst, P), jnp.float32),
            pltpu.SemaphoreType.DMA,
        ],
    )
    def k(zflat, edflat, tT_hbm, tf_hbm, src_hbm, dst_hbm, zeros_hbm, out,
          rbuf, srcb, sidx, dstb, tfb, edb, tTh, wbuf, agg, sem):
        c = lax.axis_index("c")
        s = lax.axis_index("s")
        tile = s * NCORES + c
        base = tile * EPT
        iota = lax.iota(jnp.int32, 16)

        def head_body(h, carry0):
            plsc.subcore_barrier()

            @pl.when(s == 0)
            def _zero():
                pltpu.sync_copy(zeros_hbm, agg)

            pltpu.sync_copy(edflat.at[pl.ds(h * n_dst, n_dst)], edb)
            pltpu.sync_copy(tT_hbm.at[pl.ds(h * 16, 16)], tTh)
            plsc.subcore_barrier()

            def block_body(b, carry):
                eb = base + b * NB
                pltpu.sync_copy(src_hbm.at[pl.ds(eb, NB)], srcb)
                pltpu.sync_copy(dst_hbm.at[pl.ds(eb, NB)], dstb)
                pltpu.sync_copy(tf_hbm.at[pl.ds(eb, NB)], tfb)
                for j in range(NSUBS):
                    for kk in range(NSUB // 16):
                        off = j * NSUB + kk * 16
                        sidx[j, pl.ds(kk * 16, 16)] = dstb[pl.ds(off, 16)]
                descs = [
                    pltpu.async_copy(
                        zflat.at[h].at[srcb.at[pl.ds(j * NSUB, NSUB)]],
                        rbuf.at[pl.ds(j * NSUB, NSUB)], sem)
                    for j in range(NSUBS)
                ]
                for d in descs:
                    d.wait()

                def group_static(ebase):
                    esv = plsc.load_gather(
                        rbuf, [ebase + iota,
                               jnp.full((16,), dh + 1, jnp.int32)])
                    edv = plsc.load_gather(edb, [dstb[pl.ds(ebase, 16)]])
                    efv = plsc.load_gather(tTh, [tfb[pl.ds(ebase, 16)]])
                    e = esv + edv + efv
                    e = jnp.maximum(e, 0.2 * e)
                    wbuf[...] = jnp.exp(e)
                    for i in range(16):
                        wb = plsc.load_gather(wbuf, [jnp.full((16,), i, jnp.int32)])
                        row = jnp.full((16,), ebase + i, jnp.int32)
                        for v in range(P // 16):
                            col = iota + (v * 16)
                            val = plsc.load_gather(rbuf, [row, col]) * wb
                            plsc.store_scatter(rbuf, [row, col], val)

                for g in range(NG):
                    group_static(g * 16)
                for j in range(NSUBS):
                    pltpu.sync_copy(rbuf.at[pl.ds(j * NSUB, NSUB)],
                                    agg.at[sidx.at[j]], add=True)
                return carry

            lax.fori_loop(0, NBLK, block_body, 0)
            plsc.subcore_barrier()

            @pl.when(s == 0)
            def _dump():
                pltpu.sync_copy(agg, out.at[h, c])

            return carry0

        lax.fori_loop(0, nh, head_body, 0)
        plsc.subcore_barrier()

    return k


def _gat_sc(x_src, x_dst, src_idx, dst_idx, tffrac, tf_embed, Wsrc, Wdst,
            a_src, a_dst, We, nh, n_dst, P):
    n_src = x_src.shape[0]
    dh = Wsrc.shape[1] // nh
    z3 = (x_src @ Wsrc).reshape(n_src, nh, dh)
    es = jnp.sum(z3 * a_src[None], axis=-1)
    zd = (x_dst @ Wdst).reshape(n_dst, nh, dh)
    ed = jnp.sum(zd * a_dst[None], axis=-1)
    zt = jnp.transpose(z3, (1, 0, 2))
    ones = jnp.ones((nh, n_src, 1), jnp.float32)
    est = jnp.transpose(es)[:, :, None]
    pad = jnp.zeros((nh, n_src, P - dh - 2), jnp.float32)
    zflat = jnp.concatenate([zt, ones, est, pad], axis=-1)
    edflat = jnp.transpose(ed).reshape(-1)
    tT = jnp.zeros((nh, 16), jnp.float32).at[:, :10].set(
        jnp.transpose(tf_embed @ We)).reshape(-1)
    zeros = jnp.zeros((n_dst, P), jnp.float32)
    out = _edge_gat_kernel(n_src, n_dst, nh, dh, P, 256)(
        zflat, edflat, tT, tffrac, src_idx, dst_idx, zeros)
    aggs = out[:, 0] + out[:, 1]
    norm = aggs[:, :, :dh] / (aggs[:, :, dh:dh + 1] + 1e-9)
    agg = jnp.transpose(norm, (1, 0, 2)).reshape(n_dst, nh * dh)
    return jax.nn.elu(agg)


def kernel(word_ids, sent_words, sent_position, edge_src, edge_dst, tffrac,
           embed_table, tf_embed,
           conv_w3, conv_b3, conv_w4, conv_b4, conv_w5, conv_b5,
           cnn_proj_w, cnn_proj_b,
           lstm_wih_f, lstm_whh_f, lstm_b_f,
           lstm_wih_b, lstm_whh_b, lstm_b_b,
           lstm_proj_w, lstm_proj_b,
           nfp_w,
           w2s_src_w, w2s_dst_w, w2s_a_src, w2s_a_dst, w2s_edge_w,
           w2s_ffn1_w, w2s_ffn1_b, w2s_ffn2_w, w2s_ffn2_b,
           s2w_src_w, s2w_dst_w, s2w_a_src, s2w_a_dst, s2w_edge_w,
           s2w_ffn1_w, s2w_ffn1_b, s2w_ffn2_w, s2w_ffn2_b):
    word_feature = jnp.take(embed_table, word_ids, axis=0)
    sw_emb = jnp.take(embed_table, sent_words.reshape(-1), axis=0).reshape(NS, L, EMB)
    outs = []
    for k, cw, cb in ((3, conv_w3, conv_b3), (4, conv_w4, conv_b4), (5, conv_w5, conv_b5)):
        y = jax.lax.conv_general_dilated(sw_emb, cw, (1,), 'VALID', dimension_numbers=('NWC', 'WIO', 'NWC'))
        outs.append(jnp.max(jax.nn.relu(y + cb), axis=1))
    ngram = jnp.concatenate(outs, axis=-1)
    pos_tab = _pos_table()
    cnn_feature = (ngram + pos_tab[sent_position]) @ cnn_proj_w + cnn_proj_b
    seq = ngram.reshape(NDOC, SPD, EMB)
    hf = _lstm_dir(seq, lstm_wih_f, lstm_whh_f, lstm_b_f, False)
    hb = _lstm_dir(seq, lstm_wih_b, lstm_whh_b, lstm_b_b, True)
    lstm_feature = jnp.concatenate([hf, hb], axis=-1).reshape(NS, 2 * LSTMH) @ lstm_proj_w + lstm_proj_b
    sent_feature = jnp.concatenate([cnn_feature, lstm_feature], axis=-1) @ nfp_w

    def w2s(ws_, ss_):
        h = _gat_sc(ws_, ss_, edge_src, edge_dst, tffrac, tf_embed,
                    w2s_src_w, w2s_dst_w, w2s_a_src, w2s_a_dst, w2s_edge_w,
                    8, NS, 32)
        return h + jax.nn.relu(h @ w2s_ffn1_w + w2s_ffn1_b) @ w2s_ffn2_w + w2s_ffn2_b

    def s2w(ws_, ss_):
        h = _gat_sc(ss_, ws_, edge_dst, edge_src, tffrac, tf_embed,
                    s2w_src_w, s2w_dst_w, s2w_a_src, s2w_a_dst, s2w_edge_w,
                    6, NW, 64)
        return h + jax.nn.relu(h @ s2w_ffn1_w + s2w_ffn1_b) @ s2w_ffn2_w + s2w_ffn2_b

    word_state = word_feature
    sent_state = w2s(word_feature, sent_feature)
    for _ in range(NITER):
        word_state = s2w(word_state, sent_state)
        sent_state = w2s(word_state, sent_state)
    return sent_state
